# baseline (speedup 1.0000x reference)
import functools

import jax
import jax.numpy as jnp
from jax import lax
from jax.experimental import pallas as pl
from jax.experimental.pallas import tpu as pltpu
from jax.experimental.pallas import tpu_sc as plsc

_N_TABLES = 8
_VOCAB = 100000
_DIM = 64
_BATCH = 4096
_TOTAL = _N_TABLES * _BATCH

_info = plsc.get_sparse_core_info()
_NC, _NS, _L = _info.num_cores, _info.num_subcores, _info.num_lanes
_NW = _NC * _NS
_W_PER_TABLE = _NW // _N_TABLES
_D_PER_W = _DIM // _W_PER_TABLE

_SPLIT = 50048
_HI = _VOCAB - _SPLIT
_NCHUNK = _BATCH // _L


@functools.partial(
    pl.kernel,
    out_type=jax.ShapeDtypeStruct((_DIM, _TOTAL), jnp.float32),
    mesh=plsc.VectorSubcoreMesh(core_axis_name="c", subcore_axis_name="s"),
    scratch_types=[
        pltpu.VMEM((_BATCH,), jnp.int32),
        pltpu.VMEM((_BATCH + _L,), jnp.int32),
        pltpu.VMEM((_BATCH + _L,), jnp.int32),
        pltpu.VMEM((_SPLIT,), jnp.float32),
        pltpu.VMEM((_HI,), jnp.float32),
        pltpu.VMEM((_BATCH,), jnp.float32),
        pltpu.VMEM((_BATCH,), jnp.float32),
        pltpu.SemaphoreType.DMA,
        pltpu.SemaphoreType.DMA,
        pltpu.SemaphoreType.DMA,
        pltpu.SemaphoreType.DMA,
    ],
    compiler_params=pltpu.CompilerParams(
        use_tc_tiling_on_sc=True, needs_layout_passes=False
    ),
)
def _gather_kernel(
    ids_hbm, tables_hbm, out_hbm,
    ids_v, idx_v, pos_v, lo_v, hi_v, out0_v, out1_v,
    sem_lo, sem_hi, sem_o0, sem_o1,
):
    wid = lax.axis_index("s") * _NC + lax.axis_index("c")
    t = wid // _W_PER_TABLE
    d0 = (wid % _W_PER_TABLE) * _D_PER_W

    pltpu.sync_copy(ids_hbm.at[t], ids_v)

    def _part_lo(i, n):
        idx16 = ids_v[pl.ds(i * _L, _L)]
        pos16 = lax.iota(jnp.int32, _L) + i * _L
        m = idx16 < _SPLIT
        plsc.store_compressed(idx_v.at[pl.ds(n, _L)], idx16, mask=m)
        plsc.store_compressed(pos_v.at[pl.ds(n, _L)], pos16, mask=m)
        return n + jnp.sum(m.astype(jnp.int32))

    n_lo = lax.fori_loop(0, _NCHUNK, _part_lo, jnp.int32(0))

    def _part_hi(i, n):
        idx16 = ids_v[pl.ds(i * _L, _L)]
        pos16 = lax.iota(jnp.int32, _L) + i * _L
        m = idx16 >= _SPLIT
        plsc.store_compressed(idx_v.at[pl.ds(n, _L)], idx16 - _SPLIT, mask=m)
        plsc.store_compressed(pos_v.at[pl.ds(n, _L)], pos16, mask=m)
        return n + jnp.sum(m.astype(jnp.int32))

    lax.fori_loop(0, _NCHUNK, _part_hi, n_lo)

    i_bnd = n_lo // _L

    def _stage_lo(k):
        return pltpu.async_copy(
            tables_hbm.at[t, d0 + k, pl.ds(0, _SPLIT)], lo_v, sem_lo
        )

    def _stage_hi(k):
        return pltpu.async_copy(
            tables_hbm.at[t, d0 + k, pl.ds(_SPLIT, _HI)], hi_v, sem_hi
        )

    def _gather_range(col, out_v, lo, hi):
        def _body(i):
            idx16 = idx_v[pl.ds(i * _L, _L)]
            pos16 = pos_v[pl.ds(i * _L, _L)]
            vals = plsc.load_gather(col, [idx16])
            plsc.store_scatter(out_v, [pos16], vals)

        plsc.parallel_loop(lo, hi, 1, unroll=8)(_body)

    def _gather_boundary(col, out_v, below):
        @pl.when(i_bnd < _NCHUNK)
        def _():
            idx16 = idx_v[pl.ds(i_bnd * _L, _L)]
            pos16 = pos_v[pl.ds(i_bnd * _L, _L)]
            lane = lax.iota(jnp.int32, _L) + i_bnd * _L
            m = (lane < n_lo) if below else (lane >= n_lo)
            vals = plsc.load_gather(col, [idx16], mask=m)
            plsc.store_scatter(out_v, [pos16], vals, mask=m)

    out_bufs = (out0_v, out1_v)
    out_sems = (sem_o0, sem_o1)
    out_copies = [None, None]

    cp_lo = _stage_lo(0)
    for k in range(_D_PER_W):
        out_v = out_bufs[k % 2]
        if out_copies[k % 2] is not None:
            out_copies[k % 2].wait()
        cp_lo.wait()
        cp_hi = _stage_hi(k)
        _gather_range(lo_v, out_v, 0, i_bnd)
        _gather_boundary(lo_v, out_v, True)
        cp_hi.wait()
        if k + 1 < _D_PER_W:
            cp_lo = _stage_lo(k + 1)
        _gather_boundary(hi_v, out_v, False)
        _gather_range(hi_v, out_v, i_bnd + 1, _NCHUNK)
        out_copies[k % 2] = pltpu.async_copy(
            out_v, out_hbm.at[d0 + k, pl.ds(t * _BATCH, _BATCH)], out_sems[k % 2]
        )
    for c in out_copies:
        c.wait()


def kernel(ids, tables):
    tables_t = tables.transpose(0, 2, 1)
    out_t = _gather_kernel(ids, tables_t)
    return out_t.T


# speedup vs baseline: 1.0736x; 1.0736x over previous
import functools

import jax
import jax.numpy as jnp
from jax import lax
from jax.experimental import pallas as pl
from jax.experimental.pallas import tpu as pltpu
from jax.experimental.pallas import tpu_sc as plsc

_N_TABLES = 8
_VOCAB = 100000
_DIM = 64
_BATCH = 4096
_TOTAL = _N_TABLES * _BATCH

_info = plsc.get_sparse_core_info()
_NC, _NS, _L = _info.num_cores, _info.num_subcores, _info.num_lanes
_NW = _NC * _NS
_W_PER_TABLE = _NW // _N_TABLES
_D_PER_W = _DIM // _W_PER_TABLE

_SPLIT = 50048
_HI = _VOCAB - _SPLIT
_NCHUNK = _BATCH // _L


@functools.partial(
    pl.kernel,
    out_type=jax.ShapeDtypeStruct((_DIM, _TOTAL), jnp.float32),
    mesh=plsc.VectorSubcoreMesh(core_axis_name="c", subcore_axis_name="s"),
    scratch_types=[
        pltpu.VMEM((_BATCH,), jnp.int32),
        pltpu.VMEM((_BATCH + _L,), jnp.int32),
        pltpu.VMEM((_BATCH + _L,), jnp.int32),
        pltpu.VMEM((_SPLIT,), jnp.float32),
        pltpu.VMEM((_HI,), jnp.float32),
        pltpu.VMEM((_BATCH,), jnp.float32),
        pltpu.VMEM((_BATCH,), jnp.float32),
        pltpu.SemaphoreType.DMA,
        pltpu.SemaphoreType.DMA,
        pltpu.SemaphoreType.DMA,
        pltpu.SemaphoreType.DMA,
    ],
    compiler_params=pltpu.CompilerParams(
        use_tc_tiling_on_sc=True, needs_layout_passes=False
    ),
)
def _gather_kernel(
    ids_hbm, tables_hbm, out_hbm,
    ids_v, idx_v, pos_v, lo_v, hi_v, out0_v, out1_v,
    sem_lo, sem_hi, sem_o0, sem_o1,
):
    wid = lax.axis_index("s") * _NC + lax.axis_index("c")
    t = wid // _W_PER_TABLE
    d0 = (wid % _W_PER_TABLE) * _D_PER_W

    def _stage_lo(k):
        return pltpu.async_copy(
            tables_hbm.at[t, d0 + k, pl.ds(0, _SPLIT)], lo_v, sem_lo
        )

    def _stage_hi(k):
        return pltpu.async_copy(
            tables_hbm.at[t, d0 + k, pl.ds(_SPLIT, _HI)], hi_v, sem_hi
        )

    pltpu.sync_copy(ids_hbm.at[t], ids_v)

    cp_lo = _stage_lo(0)
    cp_hi = _stage_hi(0)

    def _part(i, carry):
        n_lo, n_hi_base = carry
        idx16 = ids_v[pl.ds(i * _L, _L)]
        pos16 = lax.iota(jnp.int32, _L) + i * _L
        m_lo = idx16 < _SPLIT
        c_lo = jnp.sum(m_lo.astype(jnp.int32))
        plsc.store_compressed(idx_v.at[pl.ds(n_lo, _L)], idx16, mask=m_lo)
        plsc.store_compressed(pos_v.at[pl.ds(n_lo, _L)], pos16, mask=m_lo)
        m_hi = ~m_lo
        base = n_hi_base - (_L - c_lo)
        plsc.store_compressed(idx_v.at[pl.ds(base, _L)], idx16 - _SPLIT, mask=m_hi)
        plsc.store_compressed(pos_v.at[pl.ds(base, _L)], pos16, mask=m_hi)
        return n_lo + c_lo, base

    n_lo, _unused = lax.fori_loop(
        0, _NCHUNK, _part, (jnp.int32(0), jnp.int32(_BATCH))
    )
    i_bnd = n_lo // _L

    def _gather_range(col, out_v, lo, hi):
        def _body(i):
            idx16 = idx_v[pl.ds(i * _L, _L)]
            pos16 = pos_v[pl.ds(i * _L, _L)]
            vals = plsc.load_gather(col, [idx16])
            plsc.store_scatter(out_v, [pos16], vals)

        plsc.parallel_loop(lo, hi, 1, unroll=8)(_body)

    def _gather_boundary(col, out_v, below, clamp):
        @pl.when(i_bnd < _NCHUNK)
        def _():
            idx16 = idx_v[pl.ds(i_bnd * _L, _L)]
            pos16 = pos_v[pl.ds(i_bnd * _L, _L)]
            lane = lax.iota(jnp.int32, _L) + i_bnd * _L
            m = (lane < n_lo) if below else (lane >= n_lo)
            vals = plsc.load_gather(col, [jnp.minimum(idx16, clamp)], mask=m)
            plsc.store_scatter(out_v, [pos16], vals, mask=m)

    out_bufs = (out0_v, out1_v)
    out_sems = (sem_o0, sem_o1)
    out_copies = [None, None]

    for k in range(_D_PER_W):
        out_v = out_bufs[k % 2]
        if out_copies[k % 2] is not None:
            out_copies[k % 2].wait()
        cp_lo.wait()
        _gather_range(lo_v, out_v, 0, i_bnd)
        _gather_boundary(lo_v, out_v, True, _SPLIT - 1)
        if k + 1 < _D_PER_W:
            cp_lo = _stage_lo(k + 1)
        cp_hi.wait()
        _gather_boundary(hi_v, out_v, False, _HI - 1)
        _gather_range(hi_v, out_v, i_bnd + 1, _NCHUNK)
        if k + 1 < _D_PER_W:
            cp_hi = _stage_hi(k + 1)
        out_copies[k % 2] = pltpu.async_copy(
            out_v, out_hbm.at[d0 + k, pl.ds(t * _BATCH, _BATCH)], out_sems[k % 2]
        )
    for c in out_copies:
        c.wait()


def kernel(ids, tables):
    tables_t = tables.transpose(0, 2, 1)
    out_t = _gather_kernel(ids, tables_t)
    return out_t.T


# speedup vs baseline: 1.1061x; 1.0303x over previous
import functools

import jax
import jax.numpy as jnp
from jax import lax
from jax.experimental import pallas as pl
from jax.experimental.pallas import tpu as pltpu
from jax.experimental.pallas import tpu_sc as plsc

_N_TABLES = 8
_VOCAB = 100000
_DIM = 64
_BATCH = 4096
_TOTAL = _N_TABLES * _BATCH

_info = plsc.get_sparse_core_info()
_NC, _NS, _L = _info.num_cores, _info.num_subcores, _info.num_lanes
_NW = _NC * _NS
_W_PER_TABLE = _NW // _N_TABLES
_D_PER_W = _DIM // _W_PER_TABLE

_SPLIT = 50048
_HI = _VOCAB - _SPLIT
_NCHUNK = _BATCH // _L


@functools.partial(
    pl.kernel,
    out_type=jax.ShapeDtypeStruct((_DIM, _TOTAL), jnp.float32),
    mesh=plsc.VectorSubcoreMesh(core_axis_name="c", subcore_axis_name="s"),
    scratch_types=[
        pltpu.VMEM((_BATCH,), jnp.int32),
        pltpu.VMEM((_SPLIT,), jnp.float32),
        pltpu.VMEM((_HI,), jnp.float32),
        pltpu.VMEM((_BATCH,), jnp.float32),
        pltpu.VMEM((_BATCH,), jnp.float32),
        pltpu.SemaphoreType.DMA,
        pltpu.SemaphoreType.DMA,
        pltpu.SemaphoreType.DMA,
        pltpu.SemaphoreType.DMA,
    ],
    compiler_params=pltpu.CompilerParams(
        use_tc_tiling_on_sc=True, needs_layout_passes=False
    ),
)
def _gather_kernel(
    ids_hbm, tables_hbm, out_hbm,
    ids_v, lo_v, hi_v, out0_v, out1_v,
    sem_lo, sem_hi, sem_o0, sem_o1,
):
    wid = lax.axis_index("s") * _NC + lax.axis_index("c")
    t = wid // _W_PER_TABLE
    d0 = (wid % _W_PER_TABLE) * _D_PER_W

    def _stage_lo(k):
        return pltpu.async_copy(
            tables_hbm.at[t, d0 + k, pl.ds(0, _SPLIT)], lo_v, sem_lo
        )

    def _stage_hi(k):
        return pltpu.async_copy(
            tables_hbm.at[t, d0 + k, pl.ds(_SPLIT, _HI)], hi_v, sem_hi
        )

    cp_lo = _stage_lo(0)
    cp_hi = _stage_hi(0)
    pltpu.sync_copy(ids_hbm.at[t], ids_v)

    def _pass_lo(out_v):
        def _body(i):
            idx16 = ids_v[pl.ds(i * _L, _L)]
            j = jnp.minimum(idx16, _SPLIT - 1)
            out_v[pl.ds(i * _L, _L)] = plsc.load_gather(lo_v, [j])

        plsc.parallel_loop(0, _NCHUNK, 1, unroll=8)(_body)

    def _pass_hi(out_v):
        def _body(i):
            sl = pl.ds(i * _L, _L)
            idx16 = ids_v[sl]
            j = jnp.maximum(idx16 - _SPLIT, 0)
            vals_hi = plsc.load_gather(hi_v, [j])
            out_v[sl] = jnp.where(idx16 < _SPLIT, out_v[sl], vals_hi)

        plsc.parallel_loop(0, _NCHUNK, 1, unroll=8)(_body)

    out_bufs = (out0_v, out1_v)
    out_sems = (sem_o0, sem_o1)
    out_copies = [None, None]

    for k in range(_D_PER_W):
        out_v = out_bufs[k % 2]
        if out_copies[k % 2] is not None:
            out_copies[k % 2].wait()
        cp_lo.wait()
        _pass_lo(out_v)
        if k + 1 < _D_PER_W:
            cp_lo = _stage_lo(k + 1)
        cp_hi.wait()
        _pass_hi(out_v)
        if k + 1 < _D_PER_W:
            cp_hi = _stage_hi(k + 1)
        out_copies[k % 2] = pltpu.async_copy(
            out_v, out_hbm.at[d0 + k, pl.ds(t * _BATCH, _BATCH)], out_sems[k % 2]
        )
    for c in out_copies:
        c.wait()


def kernel(ids, tables):
    tables_t = tables.transpose(0, 2, 1)
    out_t = _gather_kernel(ids, tables_t)
    return out_t.T
